# Initial kernel scaffold; baseline (speedup 1.0000x reference)
import functools

import jax
import jax.numpy as jnp
from jax import lax
from jax.experimental import pallas as pl
from jax.experimental.pallas import tpu as pltpu
from jax.experimental.pallas import tpu_sc as plsc

N = 10000
E = 320000
D = 128
OUT = 128

NC = 2
NS = 16
CH = 128
NP = 10240
RPT = NP // NS
NCHUNK = 157
EPT = NCHUNK * CH
PE = EPT * NS
BR = 1024


_mesh = plsc.VectorSubcoreMesh(core_axis_name="c", subcore_axis_name="s")


@functools.partial(
    pl.kernel,
    out_type=(
        jax.ShapeDtypeStruct((NC, NP, D), jnp.float32),
        jax.ShapeDtypeStruct((NC, NP, 16), jnp.float32),
    ),
    mesh=_mesh,
    scratch_types=[
        pltpu.VMEM_SHARED((NP, D), jnp.float32),
        pltpu.VMEM_SHARED((NP, 16), jnp.float32),
        pltpu.VMEM((CH,), jnp.int32),
        pltpu.VMEM((CH,), jnp.int32),
        pltpu.VMEM((CH, D), jnp.float32),
        pltpu.VMEM((CH, 16), jnp.float32),
        pltpu.SemaphoreType.DMA,
    ],
)
def _sc_aggregate(feat_hbm, ei_hbm, z128_hbm, z16_hbm, sums_hbm, degs_hbm,
                  acc, deg, gidx, sidx, rows, ones, sem):
    d = lax.axis_index("c")
    s = lax.axis_index("s")
    rb = s * RPT

    pltpu.sync_copy(z128_hbm, acc.at[pl.ds(rb, RPT)])
    pltpu.sync_copy(z16_hbm, deg.at[pl.ds(rb, RPT)])

    def fill_ones(i, carry):
        ones[i] = jnp.ones((16,), jnp.float32)
        return carry
    lax.fori_loop(0, CH, fill_ones, 0)
    plsc.subcore_barrier()

    ebase = s * EPT

    def step(i, carry):
        b = ebase + i * CH
        pltpu.sync_copy(ei_hbm.at[d, pl.ds(b, CH)], gidx)
        pltpu.sync_copy(ei_hbm.at[1 - d, pl.ds(b, CH)], sidx)
        pltpu.async_copy(feat_hbm.at[gidx], rows, sem).wait()
        pltpu.sync_copy(rows, acc.at[sidx], add=True)
        pltpu.sync_copy(ones, deg.at[sidx], add=True)
        return carry
    lax.fori_loop(0, NCHUNK, step, 0)
    plsc.subcore_barrier()

    pltpu.sync_copy(acc.at[pl.ds(rb, RPT)], sums_hbm.at[d, pl.ds(rb, RPT)])
    pltpu.sync_copy(deg.at[pl.ds(rb, RPT)], degs_hbm.at[d, pl.ds(rb, RPT)])


def _combine_body(feat_ref, sums_ref, degs_ref, w_ref, out_ref):
    w = w_ref[...]
    f = feat_ref[...]
    s0 = sums_ref[0]
    s1 = sums_ref[1]
    d0 = jnp.maximum(degs_ref[0][:, 0:1], 1.0)
    d1 = jnp.maximum(degs_ref[1][:, 0:1], 1.0)
    acc = jnp.dot(f, w[0:D], preferred_element_type=jnp.float32)
    acc = acc + jnp.dot(s0 / d0, w[D:2 * D], preferred_element_type=jnp.float32)
    acc = acc + jnp.dot(s1 / d1, w[2 * D:3 * D], preferred_element_type=jnp.float32)
    out_ref[...] = acc


def kernel(feat, edge_index, W):
    featp = jnp.zeros((NP, D), jnp.float32).at[:N].set(feat)
    pad = jnp.full((2, PE - E), N, jnp.int32)
    eip = jnp.concatenate([edge_index, pad], axis=1)
    z128 = jnp.zeros((RPT, D), jnp.float32)
    z16 = jnp.zeros((RPT, 16), jnp.float32)

    sums, degs = _sc_aggregate(featp, eip, z128, z16)

    outp = pl.pallas_call(
        _combine_body,
        grid=(NP // BR,),
        in_specs=[
            pl.BlockSpec((BR, D), lambda j: (j, 0)),
            pl.BlockSpec((NC, BR, D), lambda j: (0, j, 0)),
            pl.BlockSpec((NC, BR, 16), lambda j: (0, j, 0)),
            pl.BlockSpec((3 * D, OUT), lambda j: (0, 0)),
        ],
        out_specs=pl.BlockSpec((BR, OUT), lambda j: (j, 0)),
        out_shape=jax.ShapeDtypeStruct((NP, OUT), jnp.float32),
    )(featp, sums, degs, W)
    return outp[:N]


# speedup vs baseline: 4.7860x; 4.7860x over previous
import functools

import jax
import jax.numpy as jnp
from jax import lax
from jax.experimental import pallas as pl
from jax.experimental.pallas import tpu as pltpu
from jax.experimental.pallas import tpu_sc as plsc

N = 10000
E = 320000
D = 128
OUT = 128

NC = 2
NS = 16
L = 16
D2 = D + L
CH = 128
NP = 10240
RPT = NP // NS
KWB = RPT // CH
NCHUNK = 157
EPT = NCHUNK * CH
PE = EPT * NS
BR = 1024


_mesh = plsc.VectorSubcoreMesh(core_axis_name="c", subcore_axis_name="s")


@functools.partial(
    pl.kernel,
    out_type=jax.ShapeDtypeStruct((NC * NP, D2), jnp.float32),
    mesh=_mesh,
    scratch_types=[
        pltpu.VMEM_SHARED((NP, D2), jnp.float32),
        pltpu.VMEM((CH,), jnp.int32),
        pltpu.VMEM((CH,), jnp.int32),
        pltpu.VMEM((CH, D2), jnp.float32),
        pltpu.SemaphoreType.DMA,
    ],
    compiler_params=pltpu.CompilerParams(use_tc_tiling_on_sc=False),
)
def _sc_aggregate(feat_hbm, ei_hbm, sums_hbm, acc, gidx, sidx, rows, sem):
    d = lax.axis_index("c")
    s = lax.axis_index("s")
    rb = s * RPT

    def zrow(r, carry):
        def zcol(c, carry2):
            rows[r, pl.ds(c * L, L)] = jnp.zeros((L,), jnp.float32)
            return carry2
        lax.fori_loop(0, D2 // L, zcol, 0)
        return carry
    lax.fori_loop(0, CH, zrow, 0)
    for k in range(KWB):
        pltpu.sync_copy(rows, acc.at[pl.ds(rb + k * CH, CH)])
    plsc.subcore_barrier()

    gbase = d * (2 * PE) + s * EPT
    scbase = d * (2 * PE) + PE + s * EPT

    def step(i, carry):
        o = i * CH
        pltpu.sync_copy(ei_hbm.at[pl.ds(gbase + o, CH)], gidx)
        pltpu.sync_copy(ei_hbm.at[pl.ds(scbase + o, CH)], sidx)
        pltpu.async_copy(feat_hbm.at[gidx], rows, sem).wait()
        pltpu.sync_copy(rows, acc.at[sidx], add=True)
        return carry
    lax.fori_loop(0, NCHUNK, step, 0)
    plsc.subcore_barrier()

    ob = d * NP + rb
    for k in range(KWB):
        pltpu.sync_copy(acc.at[pl.ds(rb + k * CH, CH)], rows)
        pltpu.sync_copy(rows, sums_hbm.at[pl.ds(ob + k * CH, CH)])


def _combine_body(feat_ref, sums_ref, w_ref, out_ref):
    w = w_ref[...]
    f = feat_ref[...][:, 0:D]
    s0 = sums_ref[0][:, 0:D]
    s1 = sums_ref[1][:, 0:D]
    d0 = jnp.maximum(sums_ref[0][:, D:D + 1], 1.0)
    d1 = jnp.maximum(sums_ref[1][:, D:D + 1], 1.0)
    acc = jnp.dot(f, w[0:D], preferred_element_type=jnp.float32)
    acc = acc + jnp.dot(s0 / d0, w[D:2 * D], preferred_element_type=jnp.float32)
    acc = acc + jnp.dot(s1 / d1, w[2 * D:3 * D], preferred_element_type=jnp.float32)
    out_ref[...] = acc


def kernel(feat, edge_index, W):
    featp = jnp.zeros((NP, D2), jnp.float32)
    featp = featp.at[:N, :D].set(feat)
    featp = featp.at[:, D].set(1.0)
    pad = jnp.full((2, PE - E), N, jnp.int32)
    eip = jnp.concatenate([edge_index, pad], axis=1)
    src, dst = eip[0], eip[1]
    ei_flat = jnp.concatenate([src, dst, dst, src])

    sums_f = _sc_aggregate(featp, ei_flat)
    sums = sums_f.reshape(NC, NP, D2)

    outp = pl.pallas_call(
        _combine_body,
        grid=(NP // BR,),
        in_specs=[
            pl.BlockSpec((BR, D2), lambda j: (j, 0)),
            pl.BlockSpec((NC, BR, D2), lambda j: (0, j, 0)),
            pl.BlockSpec((3 * D, OUT), lambda j: (0, 0)),
        ],
        out_specs=pl.BlockSpec((BR, OUT), lambda j: (j, 0)),
        out_shape=jax.ShapeDtypeStruct((NP, OUT), jnp.float32),
    )(featp, sums, W)
    return outp[:N]
